# baseline (speedup 1.0000x reference)
import jax
import jax.numpy as jnp
from jax.experimental import pallas as pl
from jax.experimental.pallas import tpu as pltpu
from jax.experimental.pallas import tpu_sc as plsc

EMBED_DIM = 32
WINDOW = 128
GROUP = 8


def kernel(batch, table):
    n_rows, seq = batch.shape
    num_indices = n_rows * seq
    idx = batch.reshape(num_indices // WINDOW, WINDOW)

    @pl.kernel(
        out_type=jax.ShapeDtypeStruct((num_indices, EMBED_DIM), table.dtype),
        mesh=plsc.VectorSubcoreMesh(core_axis_name="c", subcore_axis_name="s"),
        scratch_types=[pltpu.SemaphoreType.DMA],
        compiler_params=pltpu.CompilerParams(use_tc_tiling_on_sc=False),
    )
    def sc_gather(table_hbm, idx_hbm, out_hbm, sem):
        def body(idx_vmem, out_vmem):
            copies = [
                pltpu.async_copy(
                    table_hbm.at[idx_vmem.at[j]],
                    out_vmem.at[pl.ds(j * WINDOW, WINDOW)],
                    sem,
                )
                for j in range(GROUP)
            ]
            for c in copies:
                c.wait()

        pltpu.emit_pipeline(
            body,
            grid=(num_indices // (WINDOW * GROUP),),
            in_specs=[pl.BlockSpec((GROUP, WINDOW), lambda i: (i, 0))],
            out_specs=[pl.BlockSpec((WINDOW * GROUP, EMBED_DIM), lambda i: (i, 0))],
            core_axis_name=("c", "s"),
            dimension_semantics=(pltpu.PARALLEL,),
        )(idx_hbm, out_hbm)

    out = sc_gather(table, idx)
    return out.reshape(n_rows, seq, EMBED_DIM)


# speedup vs baseline: 3.3839x; 3.3839x over previous
import jax
import jax.numpy as jnp
from jax import lax
from jax.experimental import pallas as pl
from jax.experimental.pallas import tpu as pltpu
from jax.experimental.pallas import tpu_sc as plsc

EMBED_DIM = 32
WINDOW = 128
NW = 32


def kernel(batch, table):
    n_rows, seq = batch.shape
    num_indices = n_rows * seq
    idx = batch.reshape(1, num_indices)
    table_rep = jnp.tile(table[None], (NW, 1, 1))

    @pl.kernel(
        out_type=jax.ShapeDtypeStruct((num_indices, EMBED_DIM), table.dtype),
        mesh=plsc.VectorSubcoreMesh(core_axis_name="c", subcore_axis_name="s"),
        compiler_params=pltpu.CompilerParams(use_tc_tiling_on_sc=False),
    )
    def sc_gather(table_hbm, idx_hbm, out_hbm):
        wid = lax.axis_index("s") * 2 + lax.axis_index("c")
        my_table = table_hbm.at[wid]

        def body(idx_vmem, out_vmem):
            pltpu.sync_copy(my_table.at[idx_vmem.at[0]], out_vmem)

        pltpu.emit_pipeline(
            body,
            grid=(num_indices // WINDOW,),
            in_specs=[pl.BlockSpec((1, WINDOW), lambda i: (0, i))],
            out_specs=[pl.BlockSpec((WINDOW, EMBED_DIM), lambda i: (i, 0))],
            core_axis_name=("c", "s"),
            dimension_semantics=(pltpu.PARALLEL,),
        )(idx_hbm, out_hbm)

    out = sc_gather(table_rep, idx)
    return out.reshape(n_rows, seq, EMBED_DIM)


# speedup vs baseline: 3.4807x; 1.0286x over previous
import jax
import jax.numpy as jnp
from jax import lax
from jax.experimental import pallas as pl
from jax.experimental.pallas import tpu as pltpu
from jax.experimental.pallas import tpu_sc as plsc

EMBED_DIM = 32
NUM_EMB = 24
WINDOW = 128
NW = 32
LANES = 16


def kernel(batch, table):
    n_rows, seq = batch.shape
    num_indices = n_rows * seq
    num_pairs = num_indices // 2
    idx = batch.reshape(1, num_indices)

    pair_tab = jnp.concatenate(
        [jnp.repeat(table, NUM_EMB, axis=0), jnp.tile(table, (NUM_EMB, 1))],
        axis=1,
    )
    pair_rep = jnp.tile(pair_tab[None], (NW, 1, 1))

    @pl.kernel(
        out_type=jax.ShapeDtypeStruct((num_pairs, 2 * EMBED_DIM), table.dtype),
        mesh=plsc.VectorSubcoreMesh(core_axis_name="c", subcore_axis_name="s"),
        scratch_types=[pltpu.VMEM((WINDOW,), jnp.int32)],
        compiler_params=pltpu.CompilerParams(
            use_tc_tiling_on_sc=False, needs_layout_passes=False
        ),
    )
    def sc_gather(tab_hbm, idx_hbm, out_hbm, pid_v):
        wid = lax.axis_index("s") * 2 + lax.axis_index("c")
        my_tab = tab_hbm.at[wid]
        zeros = jnp.zeros((LANES,), jnp.int32)
        evens = jax.lax.iota(jnp.int32, LANES) * 2

        def body(idx_vmem, out_vmem):
            for v in range(WINDOW // LANES):
                pos = evens + (2 * LANES * v)
                first = plsc.load_gather(idx_vmem, [zeros, pos])
                second = plsc.load_gather(idx_vmem, [zeros, pos + 1])
                pid_v[pl.ds(v * LANES, LANES)] = first * NUM_EMB + second
            pltpu.sync_copy(my_tab.at[pid_v], out_vmem)

        pltpu.emit_pipeline(
            body,
            grid=(num_pairs // WINDOW,),
            in_specs=[pl.BlockSpec((1, 2 * WINDOW), lambda i: (0, i))],
            out_specs=[pl.BlockSpec((WINDOW, 2 * EMBED_DIM), lambda i: (i, 0))],
            core_axis_name=("c", "s"),
            dimension_semantics=(pltpu.PARALLEL,),
        )(idx_hbm, out_hbm)

    out = sc_gather(pair_rep, idx)
    return out.reshape(n_rows, seq, EMBED_DIM)


# speedup vs baseline: 3.9654x; 1.1393x over previous
import jax
import jax.numpy as jnp
from jax import lax
from jax.experimental import pallas as pl
from jax.experimental.pallas import tpu as pltpu
from jax.experimental.pallas import tpu_sc as plsc

EMBED_DIM = 32
NUM_EMB = 24
WINDOW = 128
NW = 32
LANES = 16


def kernel(batch, table):
    n_rows, seq = batch.shape
    num_indices = n_rows * seq
    num_pairs = num_indices // 2
    idx = batch.reshape(1, num_indices)

    pair_tab = jnp.concatenate(
        [jnp.repeat(table, NUM_EMB, axis=0), jnp.tile(table, (NUM_EMB, 1))],
        axis=1,
    )
    pair_rep = jnp.tile(pair_tab[None], (NW, 1, 1))

    @pl.kernel(
        out_type=jax.ShapeDtypeStruct((num_pairs, 2 * EMBED_DIM), table.dtype),
        mesh=plsc.VectorSubcoreMesh(core_axis_name="c", subcore_axis_name="s"),
        scratch_types=[
            pltpu.VMEM((WINDOW,), jnp.int32),
            pltpu.VMEM_SHARED((NUM_EMB * NUM_EMB, 2 * EMBED_DIM), jnp.float32),
        ],
        compiler_params=pltpu.CompilerParams(
            use_tc_tiling_on_sc=False, needs_layout_passes=False
        ),
    )
    def sc_gather(tab_hbm, idx_hbm, out_hbm, pid_v, tab_v):
        wid = lax.axis_index("s") * 2 + lax.axis_index("c")
        sid = lax.axis_index("s")

        @pl.when(sid == 0)
        def _():
            pltpu.sync_copy(tab_hbm.at[wid], tab_v)

        plsc.subcore_barrier()
        zeros = jnp.zeros((LANES,), jnp.int32)
        evens = jax.lax.iota(jnp.int32, LANES) * 2

        def body(idx_vmem, out_vmem):
            for v in range(WINDOW // LANES):
                pos = evens + (2 * LANES * v)
                first = plsc.load_gather(idx_vmem, [zeros, pos])
                second = plsc.load_gather(idx_vmem, [zeros, pos + 1])
                pid_v[pl.ds(v * LANES, LANES)] = first * NUM_EMB + second
            pltpu.sync_copy(tab_v.at[pid_v], out_vmem)

        pltpu.emit_pipeline(
            body,
            grid=(num_pairs // WINDOW,),
            in_specs=[pl.BlockSpec((1, 2 * WINDOW), lambda i: (0, i))],
            out_specs=[pl.BlockSpec((WINDOW, 2 * EMBED_DIM), lambda i: (i, 0))],
            core_axis_name=("c", "s"),
            dimension_semantics=(pltpu.PARALLEL,),
        )(idx_hbm, out_hbm)

    out = sc_gather(pair_rep, idx)
    return out.reshape(n_rows, seq, EMBED_DIM)
